# baseline (speedup 1.0000x reference)
import functools
import math

import jax
import jax.numpy as jnp
from jax import lax
from jax.experimental import pallas as pl
from jax.experimental.pallas import tpu as pltpu
from jax.experimental.pallas import tpu_sc as plsc

VOCAB = 1000000
D = 64
MAX_LEN = 200
BATCH = 4096
SEQ = 200
TOTAL = BATCH * SEQ

NC = 2
NS = 16
NW = NC * NS
CH = 128
CPW = TOTAL // (NW * CH)
LANES = 16


def _positional_encoding() -> jnp.ndarray:
    w = jnp.exp(-jnp.arange(0, D, 2, dtype=jnp.float32) * math.log(10000.0) / D)
    p = jnp.arange(0, MAX_LEN, dtype=jnp.float32).reshape(MAX_LEN, 1)
    pe = jnp.zeros((MAX_LEN, D), dtype=jnp.float32)
    pe = pe.at[:, 0::2].set(jnp.sin(p * w))
    pe = pe.at[:, 1::2].set(jnp.cos(p * w))
    return pe


def _body(x_hbm, table_hbm, pe_hbm, out_hbm, idx_v, rows_v, pe_v, sem):
    wid = lax.axis_index("s") * NC + lax.axis_index("c")
    pltpu.sync_copy(pe_hbm, pe_v)

    def chunk_body(k, _):
        chunk = wid * CPW + k
        pltpu.sync_copy(x_hbm.at[chunk], idx_v)
        pltpu.async_copy(table_hbm.at[idx_v], rows_v, sem).wait()
        ph = (k * CH) % SEQ

        def row_body(r, _):
            pr = ph + r
            for j in range(D // LANES):
                sl = pl.ds(j * LANES, LANES)
                rows_v[r, sl] = rows_v[r, sl] * 8.0 + pe_v[pr, sl]
            return 0

        lax.fori_loop(0, CH, row_body, 0)
        pltpu.sync_copy(rows_v, out_hbm.at[pl.ds(chunk * CH, CH)])
        return 0

    lax.fori_loop(0, CPW, chunk_body, 0)


def kernel(x, table):
    pe = _positional_encoding()
    pe2 = jnp.concatenate([pe, pe], axis=0)
    x2 = x.reshape(TOTAL // CH, CH)

    mesh = plsc.VectorSubcoreMesh(core_axis_name="c", subcore_axis_name="s")
    k = functools.partial(
        pl.kernel,
        mesh=mesh,
        out_type=jax.ShapeDtypeStruct((TOTAL, D), jnp.float32),
        scratch_types=[
            pltpu.VMEM((CH,), jnp.int32),
            pltpu.VMEM((CH, D), jnp.float32),
            pltpu.VMEM((2 * MAX_LEN, D), jnp.float32),
            pltpu.SemaphoreType.DMA,
        ],
        compiler_params=pltpu.CompilerParams(use_tc_tiling_on_sc=False),
    )(_body)
    out = k(x2, table, pe2)
    return out.reshape(BATCH, SEQ, D)


# speedup vs baseline: 1.1845x; 1.1845x over previous
import functools
import math

import jax
import jax.numpy as jnp
from jax import lax
from jax.experimental import pallas as pl
from jax.experimental.pallas import tpu as pltpu
from jax.experimental.pallas import tpu_sc as plsc

VOCAB = 1000000
D = 64
MAX_LEN = 200
BATCH = 4096
SEQ = 200
TOTAL = BATCH * SEQ

NC = 2
NS = 16
NW = NC * NS
CH = 128
CPW = TOTAL // (NW * CH)
NB = 2
LANES = 16


def _positional_encoding() -> jnp.ndarray:
    w = jnp.exp(-jnp.arange(0, D, 2, dtype=jnp.float32) * math.log(10000.0) / D)
    p = jnp.arange(0, MAX_LEN, dtype=jnp.float32).reshape(MAX_LEN, 1)
    pe = jnp.zeros((MAX_LEN, D), dtype=jnp.float32)
    pe = pe.at[:, 0::2].set(jnp.sin(p * w))
    pe = pe.at[:, 1::2].set(jnp.cos(p * w))
    return pe


def _body(x_hbm, table_hbm, pe_hbm, out_hbm,
          idx_v, rows0, rows1, out0, out1, pe_v,
          gs0, gs1, ws0, ws1):
    wid = lax.axis_index("s") * NC + lax.axis_index("c")
    rows = (rows0, rows1)
    outs = (out0, out1)
    gsem = (gs0, gs1)
    wsem = (ws0, ws1)

    pltpu.sync_copy(pe_hbm, pe_v)
    pltpu.sync_copy(x_hbm.at[wid], idx_v)

    for b in range(NB):
        pltpu.async_copy(table_hbm.at[idx_v.at[b]], rows[b], gsem[b])

    @pl.loop(0, CPW, step=NB)
    def outer(k0):
        for b in range(NB):
            k = k0 + b
            pltpu.make_async_copy(
                table_hbm.at[idx_v.at[k]], rows[b], gsem[b]).wait()

            @pl.when(k0 > 0)
            def _wait_write():
                pltpu.make_async_copy(
                    outs[b], out_hbm.at[pl.ds(0, CH)], wsem[b]).wait()

            ph = (k * CH) % SEQ

            @pl.loop(0, CH, unroll=4)
            def _row(r):
                pr = ph + r
                for j in range(D // LANES):
                    sl = pl.ds(j * LANES, LANES)
                    outs[b][r, sl] = rows[b][r, sl] * 8.0 + pe_v[pr, sl]

            @pl.when(k < CPW - NB)
            def _next_gather():
                pltpu.async_copy(
                    table_hbm.at[idx_v.at[k + NB]], rows[b], gsem[b])

            gc = wid * CPW + k
            pltpu.async_copy(outs[b], out_hbm.at[pl.ds(gc * CH, CH)], wsem[b])

    for b in range(NB):
        pltpu.make_async_copy(outs[b], out_hbm.at[pl.ds(0, CH)], wsem[b]).wait()


def kernel(x, table):
    pe = _positional_encoding()
    pe2 = jnp.concatenate([pe, pe], axis=0)
    x2 = x.reshape(NW, CPW, CH)

    mesh = plsc.VectorSubcoreMesh(core_axis_name="c", subcore_axis_name="s")
    k = functools.partial(
        pl.kernel,
        mesh=mesh,
        out_type=jax.ShapeDtypeStruct((TOTAL, D), jnp.float32),
        scratch_types=[
            pltpu.VMEM((CPW, CH), jnp.int32),
            pltpu.VMEM((CH, D), jnp.float32),
            pltpu.VMEM((CH, D), jnp.float32),
            pltpu.VMEM((CH, D), jnp.float32),
            pltpu.VMEM((CH, D), jnp.float32),
            pltpu.VMEM((2 * MAX_LEN, D), jnp.float32),
            pltpu.SemaphoreType.DMA,
            pltpu.SemaphoreType.DMA,
            pltpu.SemaphoreType.DMA,
            pltpu.SemaphoreType.DMA,
        ],
        compiler_params=pltpu.CompilerParams(use_tc_tiling_on_sc=False),
    )(_body)
    out = k(x2, table, pe2)
    return out.reshape(BATCH, SEQ, D)
